# baseline (speedup 1.0000x reference)
import functools

import jax
import jax.numpy as jnp
from jax import lax
from jax.experimental import pallas as pl
from jax.experimental.pallas import tpu as pltpu
from jax.experimental.pallas import tpu_sc as plsc

N = 10000
E = 320000
D = 128
DH = D // 2
G = 64
NC = 2
NS = 16
CHUNK = 128
CPW = 160
E_PAD = NS * CPW * CHUNK
AGG_ROWS = 10240
ZRPT = AGG_ROWS // NS
NBUF = 4


def _sc_segment_sum(z_cat, src_c, dst_c):
    mesh = plsc.VectorSubcoreMesh(core_axis_name="c", subcore_axis_name="s")

    @functools.partial(
        pl.kernel,
        out_type=jax.ShapeDtypeStruct((AGG_ROWS, NC, DH), jnp.float32),
        mesh=mesh,
        compiler_params=pltpu.CompilerParams(use_tc_tiling_on_sc=False),
        scratch_types=[
            pltpu.VMEM((CPW, CHUNK), jnp.int32),
            pltpu.VMEM((CPW, CHUNK), jnp.int32),
            pltpu.VMEM((NBUF, CHUNK, DH), jnp.float32),
            pltpu.VMEM_SHARED((AGG_ROWS, DH), jnp.float32),
            [pltpu.SemaphoreType.DMA] * NBUF,
            [pltpu.SemaphoreType.DMA] * NBUF,
        ],
    )
    def k(z_hbm, src_hbm, dst_hbm, out_hbm, src_v, dst_v, rows_v, agg_sh,
          gsems, ssems):
        c = lax.axis_index("c")
        s = lax.axis_index("s")

        pltpu.sync_copy(src_hbm.at[c, s], src_v)
        pltpu.sync_copy(dst_hbm.at[s], dst_v)

        zero = jnp.zeros((16,), jnp.float32)

        def zrow(i, carry):
            for j in range(DH // 16):
                rows_v[0, i, pl.ds(j * 16, 16)] = zero
            return carry

        lax.fori_loop(0, CHUNK, zrow, 0)
        for off in range(0, ZRPT, CHUNK):
            pltpu.sync_copy(rows_v.at[0],
                            agg_sh.at[pl.ds(s * ZRPT + off, CHUNK)])
        plsc.subcore_barrier()

        def gather(j, b):
            pltpu.async_copy(z_hbm.at[src_v.at[j]], rows_v.at[b], gsems[b])

        def gather_wait(j, b):
            pltpu.make_async_copy(
                z_hbm.at[src_v.at[j]], rows_v.at[b], gsems[b]).wait()

        def scatter(j, b):
            pltpu.async_copy(
                rows_v.at[b], agg_sh.at[dst_v.at[j]], ssems[b], add=True)

        def scatter_wait(j, b):
            pltpu.make_async_copy(
                rows_v.at[b], agg_sh.at[dst_v.at[j]], ssems[b]).wait()

        for b in range(NBUF):
            gather(b, b)
        n_it = CPW // NBUF

        def body(i, carry):
            for b in range(NBUF):
                j = i * NBUF + b
                gather_wait(j, b)
                scatter(j, b)
                scatter_wait(j, b)

                @pl.when(i < n_it - 1)
                def _():
                    gather(j + NBUF, b)
            return carry

        lax.fori_loop(0, n_it, body, 0)
        plsc.subcore_barrier()

        for off in range(0, ZRPT, CHUNK):
            base = s * ZRPT + off
            pltpu.sync_copy(agg_sh.at[pl.ds(base, CHUNK)], rows_v.at[0])
            pltpu.sync_copy(rows_v.at[0],
                            out_hbm.at[pl.ds(base, CHUNK), c])

    return k(z_cat, src_c, dst_c)


BLK = 2000


def _tc_linear_stats(z, p, W, b, eps2d):
    def body(z_ref, p_ref, w_ref, b_ref, e_ref, h_ref, s1_ref, s2_ref):
        i = pl.program_id(0)
        pre = (1.0 + e_ref[0, 0]) * z_ref[...] + p_ref[...]
        h = lax.dot_general(pre, w_ref[...], (((1,), (1,)), ((), ())),
                            precision=lax.Precision.HIGHEST,
                            preferred_element_type=jnp.float32)
        h = jnp.maximum(h + b_ref[...], 0.0)
        h_ref[...] = h

        @pl.when(i == 0)
        def _():
            s1_ref[...] = jnp.zeros_like(s1_ref)
            s2_ref[...] = jnp.zeros_like(s2_ref)

        s1_ref[...] += jnp.sum(h, axis=0)[None]
        s2_ref[...] += jnp.sum(h * h, axis=0)[None]

    return pl.pallas_call(
        body,
        grid=(N // BLK,),
        in_specs=[
            pl.BlockSpec((BLK, D), lambda i: (i, 0)),
            pl.BlockSpec((BLK, D), lambda i: (i, 0)),
            pl.BlockSpec((D, D), lambda i: (0, 0)),
            pl.BlockSpec((1, D), lambda i: (0, 0)),
            pl.BlockSpec((1, 1), lambda i: (0, 0)),
        ],
        out_specs=[
            pl.BlockSpec((BLK, D), lambda i: (i, 0)),
            pl.BlockSpec((1, D), lambda i: (0, 0)),
            pl.BlockSpec((1, D), lambda i: (0, 0)),
        ],
        out_shape=[
            jax.ShapeDtypeStruct((N, D), jnp.float32),
            jax.ShapeDtypeStruct((1, D), jnp.float32),
            jax.ShapeDtypeStruct((1, D), jnp.float32),
        ],
    )(z, p, W, b, eps2d)


def _tc_bn_pool(h, s1, s2, gamma, beta, onehot):
    def body(h_ref, s1_ref, s2_ref, g_ref, be_ref, oh_ref,
             z_ref, gout_ref):
        i = pl.program_id(0)
        mean = s1_ref[...] / N
        var = s2_ref[...] / N - mean * mean
        invstd = lax.rsqrt(var + 1e-5)
        z = (h_ref[...] - mean) * (invstd * g_ref[...]) + be_ref[...]
        z_ref[...] = z

        @pl.when(i == 0)
        def _():
            gout_ref[...] = jnp.zeros_like(gout_ref)

        gout_ref[...] += lax.dot_general(
            oh_ref[...], z, (((0,), (0,)), ((), ())),
            precision=lax.Precision.HIGHEST,
            preferred_element_type=jnp.float32)

    return pl.pallas_call(
        body,
        grid=(N // BLK,),
        in_specs=[
            pl.BlockSpec((BLK, D), lambda i: (i, 0)),
            pl.BlockSpec((1, D), lambda i: (0, 0)),
            pl.BlockSpec((1, D), lambda i: (0, 0)),
            pl.BlockSpec((1, D), lambda i: (0, 0)),
            pl.BlockSpec((1, D), lambda i: (0, 0)),
            pl.BlockSpec((BLK, G), lambda i: (i, 0)),
        ],
        out_specs=[
            pl.BlockSpec((BLK, D), lambda i: (i, 0)),
            pl.BlockSpec((G, D), lambda i: (0, 0)),
        ],
        out_shape=[
            jax.ShapeDtypeStruct((N, D), jnp.float32),
            jax.ShapeDtypeStruct((G, D), jnp.float32),
        ],
    )(h, s1, s2, gamma, beta, onehot)


def kernel(x, edge_index, batch, W0, b0, eps0, g0, be0, W1, b1, eps1, g1, be1):
    npad = E_PAD - E
    pad_src = (jnp.arange(npad, dtype=jnp.int32) * 7) % N
    pad_dst = N + (jnp.arange(npad, dtype=jnp.int32) % (AGG_ROWS - N))
    src_p = jnp.concatenate([edge_index[0], pad_src])
    src_c = jnp.stack([2 * src_p, 2 * src_p + 1]).reshape(NC, NS, CPW, CHUNK)
    dst_c = jnp.concatenate([edge_index[1], pad_dst]).reshape(NS, CPW, CHUNK)

    onehot = (batch[:, None] == jnp.arange(G, dtype=batch.dtype)[None, :]
              ).astype(jnp.float32)

    Ws = jnp.stack([W0, W1])
    bs = jnp.stack([b0[None], b1[None]])
    epss = jnp.stack([jnp.reshape(eps0, (1, 1)), jnp.reshape(eps1, (1, 1))])
    gammas = jnp.stack([g0[None], g1[None]])
    betas = jnp.stack([be0[None], be1[None]])

    def step(z, params):
        W, b, eps2d, gamma, beta = params
        p = _sc_segment_sum(z.reshape(NC * N, DH), src_c, dst_c)
        h, s1, s2 = _tc_linear_stats(z, p.reshape(AGG_ROWS, D), W, b, eps2d)
        z_next, gr = _tc_bn_pool(h, s1, s2, gamma, beta, onehot)
        return z_next, (z_next, gr)

    _, (zs, grs) = lax.scan(step, x, (Ws, bs, epss, gammas, betas))

    z_out = jnp.concatenate([zs[0], zs[1]], axis=1)
    g_out = jnp.concatenate([grs[0], grs[1]], axis=1)
    return (z_out, g_out)


# speedup vs baseline: 1.0025x; 1.0025x over previous
import functools

import jax
import jax.numpy as jnp
from jax import lax
from jax.experimental import pallas as pl
from jax.experimental.pallas import tpu as pltpu
from jax.experimental.pallas import tpu_sc as plsc

N = 10000
E = 320000
D = 128
DH = D // 2
G = 64
NL = 2
NC = 2
NS = 16
CHUNK = 128
CPW = 160
E_PAD = NS * CPW * CHUNK
AGG_ROWS = 10240
ZRPT = AGG_ROWS // NS
NBUF = 4


def _sc_segment_sum(z_cat, src_c, dst_c):
    mesh = plsc.VectorSubcoreMesh(core_axis_name="c", subcore_axis_name="s")

    @functools.partial(
        pl.kernel,
        out_type=jax.ShapeDtypeStruct((NC, AGG_ROWS, DH), jnp.float32),
        mesh=mesh,
        compiler_params=pltpu.CompilerParams(use_tc_tiling_on_sc=False),
        scratch_types=[
            pltpu.VMEM((CPW, CHUNK), jnp.int32),
            pltpu.VMEM((CPW, CHUNK), jnp.int32),
            pltpu.VMEM((NBUF, CHUNK, DH), jnp.float32),
            pltpu.VMEM_SHARED((AGG_ROWS, DH), jnp.float32),
            [pltpu.SemaphoreType.DMA] * NBUF,
            [pltpu.SemaphoreType.DMA] * NBUF,
        ],
    )
    def k(z_hbm, src_hbm, dst_hbm, out_hbm, src_v, dst_v, rows_v, agg_sh,
          gsems, ssems):
        c = lax.axis_index("c")
        s = lax.axis_index("s")

        pltpu.sync_copy(src_hbm.at[c, s], src_v)
        pltpu.sync_copy(dst_hbm.at[s], dst_v)

        zero = jnp.zeros((16,), jnp.float32)

        def zrow(i, carry):
            for j in range(DH // 16):
                rows_v[0, i, pl.ds(j * 16, 16)] = zero
            return carry

        lax.fori_loop(0, CHUNK, zrow, 0)
        for off in range(0, ZRPT, CHUNK):
            pltpu.sync_copy(rows_v.at[0],
                            agg_sh.at[pl.ds(s * ZRPT + off, CHUNK)])
        plsc.subcore_barrier()

        def gather(j, b):
            pltpu.async_copy(z_hbm.at[src_v.at[j]], rows_v.at[b], gsems[b])

        def gather_wait(j, b):
            pltpu.make_async_copy(
                z_hbm.at[src_v.at[j]], rows_v.at[b], gsems[b]).wait()

        def scatter(j, b):
            pltpu.async_copy(
                rows_v.at[b], agg_sh.at[dst_v.at[j]], ssems[b], add=True)

        def scatter_wait(j, b):
            pltpu.make_async_copy(
                rows_v.at[b], agg_sh.at[dst_v.at[j]], ssems[b]).wait()

        for b in range(NBUF):
            gather(b, b)
        n_it = CPW // NBUF

        def body(i, carry):
            for b in range(NBUF):
                j = i * NBUF + b
                gather_wait(j, b)
                scatter(j, b)
                scatter_wait(j, b)

                @pl.when(i < n_it - 1)
                def _():
                    gather(j + NBUF, b)
            return carry

        lax.fori_loop(0, n_it, body, 0)
        plsc.subcore_barrier()

        for off in range(0, ZRPT, CHUNK):
            base = s * ZRPT + off
            pltpu.sync_copy(agg_sh.at[pl.ds(base, CHUNK)], rows_v.at[0])
            pltpu.sync_copy(rows_v.at[0], out_hbm.at[c, pl.ds(base, CHUNK)])

    return k(z_cat, src_c, dst_c)


BLK = 2000


def _tc_linear_stats(z, p, W, b, eps2d):
    def body(z_ref, p_ref, w_ref, b_ref, e_ref, h_ref, s1_ref, s2_ref):
        i = pl.program_id(0)
        zb = jnp.concatenate([z_ref[0], z_ref[1]], axis=1)
        pb = jnp.concatenate([p_ref[0], p_ref[1]], axis=1)
        pre = (1.0 + e_ref[0, 0]) * zb + pb
        h = lax.dot_general(pre, w_ref[...], (((1,), (1,)), ((), ())),
                            precision=lax.Precision.HIGHEST,
                            preferred_element_type=jnp.float32)
        h = jnp.maximum(h + b_ref[...], 0.0)
        h_ref[...] = h

        @pl.when(i == 0)
        def _():
            s1_ref[...] = jnp.zeros_like(s1_ref)
            s2_ref[...] = jnp.zeros_like(s2_ref)

        s1_ref[...] += jnp.sum(h, axis=0)[None]
        s2_ref[...] += jnp.sum(h * h, axis=0)[None]

    return pl.pallas_call(
        body,
        grid=(N // BLK,),
        in_specs=[
            pl.BlockSpec((2, BLK, DH), lambda i: (0, i, 0)),
            pl.BlockSpec((2, BLK, DH), lambda i: (0, i, 0)),
            pl.BlockSpec((D, D), lambda i: (0, 0)),
            pl.BlockSpec((1, D), lambda i: (0, 0)),
            pl.BlockSpec((1, 1), lambda i: (0, 0)),
        ],
        out_specs=[
            pl.BlockSpec((BLK, D), lambda i: (i, 0)),
            pl.BlockSpec((1, D), lambda i: (0, 0)),
            pl.BlockSpec((1, D), lambda i: (0, 0)),
        ],
        out_shape=[
            jax.ShapeDtypeStruct((N, D), jnp.float32),
            jax.ShapeDtypeStruct((1, D), jnp.float32),
            jax.ShapeDtypeStruct((1, D), jnp.float32),
        ],
    )(z, p, W, b, eps2d)


def _tc_bn(h, s1, s2, gamma, beta):
    def body(h_ref, s1_ref, s2_ref, g_ref, be_ref, z_ref):
        mean = s1_ref[...] / N
        var = s2_ref[...] / N - mean * mean
        invstd = lax.rsqrt(var + 1e-5)
        z = (h_ref[...] - mean) * (invstd * g_ref[...]) + be_ref[...]
        z_ref[0] = z[:, :DH]
        z_ref[1] = z[:, DH:]

    return pl.pallas_call(
        body,
        grid=(N // BLK,),
        in_specs=[
            pl.BlockSpec((BLK, D), lambda i: (i, 0)),
            pl.BlockSpec((1, D), lambda i: (0, 0)),
            pl.BlockSpec((1, D), lambda i: (0, 0)),
            pl.BlockSpec((1, D), lambda i: (0, 0)),
            pl.BlockSpec((1, D), lambda i: (0, 0)),
        ],
        out_specs=pl.BlockSpec((2, BLK, DH), lambda i: (0, i, 0)),
        out_shape=jax.ShapeDtypeStruct((2, N, DH), jnp.float32),
    )(h, s1, s2, gamma, beta)


def _tc_pool(zs, onehot):
    def body(z_ref, oh_ref, g_ref):
        i = pl.program_id(1)
        z = jnp.concatenate([z_ref[0, 0], z_ref[0, 1]], axis=1)

        @pl.when(i == 0)
        def _():
            g_ref[...] = jnp.zeros_like(g_ref)

        g_ref[...] += lax.dot_general(
            oh_ref[...], z, (((0,), (0,)), ((), ())),
            precision=lax.Precision.HIGHEST,
            preferred_element_type=jnp.float32)[None]

    return pl.pallas_call(
        body,
        grid=(NL, N // BLK),
        in_specs=[
            pl.BlockSpec((1, 2, BLK, DH), lambda l, i: (l, 0, i, 0)),
            pl.BlockSpec((BLK, G), lambda l, i: (i, 0)),
        ],
        out_specs=pl.BlockSpec((1, G, D), lambda l, i: (l, 0, 0)),
        out_shape=jax.ShapeDtypeStruct((NL, G, D), jnp.float32),
    )(zs, onehot)


def kernel(x, edge_index, batch, W0, b0, eps0, g0, be0, W1, b1, eps1, g1, be1):
    npad = E_PAD - E
    pad_src = (jnp.arange(npad, dtype=jnp.int32) * 7) % N
    pad_dst = N + (jnp.arange(npad, dtype=jnp.int32) % (AGG_ROWS - N))
    src_p = jnp.concatenate([edge_index[0], pad_src])
    src_c = jnp.stack([src_p, src_p + N]).reshape(NC, NS, CPW, CHUNK)
    dst_c = jnp.concatenate([edge_index[1], pad_dst]).reshape(NS, CPW, CHUNK)

    onehot = (batch[:, None] == jnp.arange(G, dtype=batch.dtype)[None, :]
              ).astype(jnp.float32)

    x_split = jnp.stack([x[:, :DH], x[:, DH:]])

    Ws = jnp.stack([W0, W1])
    bs = jnp.stack([b0[None], b1[None]])
    epss = jnp.stack([jnp.reshape(eps0, (1, 1)), jnp.reshape(eps1, (1, 1))])
    gammas = jnp.stack([g0[None], g1[None]])
    betas = jnp.stack([be0[None], be1[None]])

    def step(z, params):
        W, b, eps2d, gamma, beta = params
        p = _sc_segment_sum(z.reshape(NC * N, DH), src_c, dst_c)
        h, s1, s2 = _tc_linear_stats(z, p, W, b, eps2d)
        z_next = _tc_bn(h, s1, s2, gamma, beta)
        return z_next, z_next

    _, zs = lax.scan(step, x_split, (Ws, bs, epss, gammas, betas))

    gs = _tc_pool(zs, onehot)

    z_out = jnp.concatenate([zs[0, 0], zs[0, 1], zs[1, 0], zs[1, 1]], axis=1)
    g_out = jnp.concatenate([gs[0], gs[1]], axis=1)
    return (z_out, g_out)


# speedup vs baseline: 1.0082x; 1.0057x over previous
import functools

import jax
import jax.numpy as jnp
from jax import lax
from jax.experimental import pallas as pl
from jax.experimental.pallas import tpu as pltpu
from jax.experimental.pallas import tpu_sc as plsc

N = 10000
E = 320000
D = 128
DH = D // 2
G = 64
NL = 2
NC = 2
NS = 16
CHUNK = 128
CPW = 160
E_PAD = NS * CPW * CHUNK
AGG_ROWS = 10240
ZRPT = AGG_ROWS // NS
NBUF = 5


def _sc_segment_sum(z_cat, src_c, dst_c):
    mesh = plsc.VectorSubcoreMesh(core_axis_name="c", subcore_axis_name="s")

    @functools.partial(
        pl.kernel,
        out_type=jax.ShapeDtypeStruct((NC, AGG_ROWS, DH), jnp.float32),
        mesh=mesh,
        compiler_params=pltpu.CompilerParams(use_tc_tiling_on_sc=False),
        scratch_types=[
            pltpu.VMEM((CPW, CHUNK), jnp.int32),
            pltpu.VMEM((CPW, CHUNK), jnp.int32),
            pltpu.VMEM((NBUF, CHUNK, DH), jnp.float32),
            pltpu.VMEM_SHARED((AGG_ROWS, DH), jnp.float32),
            [pltpu.SemaphoreType.DMA] * NBUF,
            [pltpu.SemaphoreType.DMA] * NBUF,
        ],
    )
    def k(z_hbm, src_hbm, dst_hbm, out_hbm, src_v, dst_v, rows_v, agg_sh,
          gsems, ssems):
        c = lax.axis_index("c")
        s = lax.axis_index("s")

        pltpu.sync_copy(src_hbm.at[c, s], src_v)
        pltpu.sync_copy(dst_hbm.at[s], dst_v)

        zero = jnp.zeros((16,), jnp.float32)

        def zrow(i, carry):
            for j in range(DH // 16):
                rows_v[0, i, pl.ds(j * 16, 16)] = zero
            return carry

        lax.fori_loop(0, CHUNK, zrow, 0)
        for off in range(0, ZRPT, CHUNK):
            pltpu.sync_copy(rows_v.at[0],
                            agg_sh.at[pl.ds(s * ZRPT + off, CHUNK)])
        plsc.subcore_barrier()

        def gather(j, b):
            pltpu.async_copy(z_hbm.at[src_v.at[j]], rows_v.at[b], gsems[b])

        def gather_wait(j, b):
            pltpu.make_async_copy(
                z_hbm.at[src_v.at[j]], rows_v.at[b], gsems[b]).wait()

        def scatter(j, b):
            pltpu.async_copy(
                rows_v.at[b], agg_sh.at[dst_v.at[j]], ssems[b], add=True)

        def scatter_wait(j, b):
            pltpu.make_async_copy(
                rows_v.at[b], agg_sh.at[dst_v.at[j]], ssems[b]).wait()

        for b in range(NBUF):
            gather(b, b)
        n_it = CPW // NBUF
        K = 2

        def body(i, carry):
            for b in range(NBUF):
                j = i * NBUF + b
                gather_wait(j, b)
                scatter(j, b)
                bk = (b + NBUF - K) % NBUF
                cond = (i > 0) if b < K else (i < n_it - 1)

                @pl.when(cond)
                def _():
                    scatter_wait(j - K, bk)
                    gather(j + NBUF - K, bk)
            return carry

        lax.fori_loop(0, n_it, body, 0)
        for t in range(NBUF):
            j = CPW - NBUF + t
            scatter_wait(j, j % NBUF)
        plsc.subcore_barrier()

        for off in range(0, ZRPT, CHUNK):
            base = s * ZRPT + off
            pltpu.sync_copy(agg_sh.at[pl.ds(base, CHUNK)], rows_v.at[0])
            pltpu.sync_copy(rows_v.at[0], out_hbm.at[c, pl.ds(base, CHUNK)])

    return k(z_cat, src_c, dst_c)


BLK = 2000


def _tc_linear_stats(z, p, W, b, eps2d):
    def body(z_ref, p_ref, w_ref, b_ref, e_ref, h_ref, s1_ref, s2_ref):
        i = pl.program_id(0)
        zb = jnp.concatenate([z_ref[0], z_ref[1]], axis=1)
        pb = jnp.concatenate([p_ref[0], p_ref[1]], axis=1)
        pre = (1.0 + e_ref[0, 0]) * zb + pb
        h = lax.dot_general(pre, w_ref[...], (((1,), (1,)), ((), ())),
                            precision=lax.Precision.HIGHEST,
                            preferred_element_type=jnp.float32)
        h = jnp.maximum(h + b_ref[...], 0.0)
        h_ref[...] = h

        @pl.when(i == 0)
        def _():
            s1_ref[...] = jnp.zeros_like(s1_ref)
            s2_ref[...] = jnp.zeros_like(s2_ref)

        s1_ref[...] += jnp.sum(h, axis=0)[None]
        s2_ref[...] += jnp.sum(h * h, axis=0)[None]

    return pl.pallas_call(
        body,
        grid=(N // BLK,),
        in_specs=[
            pl.BlockSpec((2, BLK, DH), lambda i: (0, i, 0)),
            pl.BlockSpec((2, BLK, DH), lambda i: (0, i, 0)),
            pl.BlockSpec((D, D), lambda i: (0, 0)),
            pl.BlockSpec((1, D), lambda i: (0, 0)),
            pl.BlockSpec((1, 1), lambda i: (0, 0)),
        ],
        out_specs=[
            pl.BlockSpec((BLK, D), lambda i: (i, 0)),
            pl.BlockSpec((1, D), lambda i: (0, 0)),
            pl.BlockSpec((1, D), lambda i: (0, 0)),
        ],
        out_shape=[
            jax.ShapeDtypeStruct((N, D), jnp.float32),
            jax.ShapeDtypeStruct((1, D), jnp.float32),
            jax.ShapeDtypeStruct((1, D), jnp.float32),
        ],
    )(z, p, W, b, eps2d)


def _tc_bn_pool(h, s1, s2, gamma, beta, onehot):
    def body(h_ref, s1_ref, s2_ref, g_ref, be_ref, oh_ref, z_ref, gout_ref):
        i = pl.program_id(0)
        mean = s1_ref[...] / N
        var = s2_ref[...] / N - mean * mean
        invstd = lax.rsqrt(var + 1e-5)
        z = (h_ref[...] - mean) * (invstd * g_ref[...]) + be_ref[...]
        z_ref[0] = z[:, :DH]
        z_ref[1] = z[:, DH:]

        @pl.when(i == 0)
        def _():
            gout_ref[...] = jnp.zeros_like(gout_ref)

        gout_ref[...] += lax.dot_general(
            oh_ref[...], z, (((0,), (0,)), ((), ())),
            precision=lax.Precision.HIGHEST,
            preferred_element_type=jnp.float32)

    return pl.pallas_call(
        body,
        grid=(N // BLK,),
        in_specs=[
            pl.BlockSpec((BLK, D), lambda i: (i, 0)),
            pl.BlockSpec((1, D), lambda i: (0, 0)),
            pl.BlockSpec((1, D), lambda i: (0, 0)),
            pl.BlockSpec((1, D), lambda i: (0, 0)),
            pl.BlockSpec((1, D), lambda i: (0, 0)),
            pl.BlockSpec((BLK, G), lambda i: (i, 0)),
        ],
        out_specs=[
            pl.BlockSpec((2, BLK, DH), lambda i: (0, i, 0)),
            pl.BlockSpec((G, D), lambda i: (0, 0)),
        ],
        out_shape=[
            jax.ShapeDtypeStruct((2, N, DH), jnp.float32),
            jax.ShapeDtypeStruct((G, D), jnp.float32),
        ],
    )(h, s1, s2, gamma, beta, onehot)


def kernel(x, edge_index, batch, W0, b0, eps0, g0, be0, W1, b1, eps1, g1, be1):
    npad = E_PAD - E
    pad_src = (jnp.arange(npad, dtype=jnp.int32) * 7) % N
    pad_dst = N + (jnp.arange(npad, dtype=jnp.int32) % (AGG_ROWS - N))
    src_p = jnp.concatenate([edge_index[0], pad_src])
    src_c = jnp.stack([src_p, src_p + N]).reshape(NC, NS, CPW, CHUNK)
    dst_c = jnp.concatenate([edge_index[1], pad_dst]).reshape(NS, CPW, CHUNK)

    onehot = (batch[:, None] == jnp.arange(G, dtype=batch.dtype)[None, :]
              ).astype(jnp.float32)

    x_split = jnp.stack([x[:, :DH], x[:, DH:]])

    Ws = jnp.stack([W0, W1])
    bs = jnp.stack([b0[None], b1[None]])
    epss = jnp.stack([jnp.reshape(eps0, (1, 1)), jnp.reshape(eps1, (1, 1))])
    gammas = jnp.stack([g0[None], g1[None]])
    betas = jnp.stack([be0[None], be1[None]])

    def step(z, params):
        W, b, eps2d, gamma, beta = params
        p = _sc_segment_sum(z.reshape(NC * N, DH), src_c, dst_c)
        h, s1, s2 = _tc_linear_stats(z, p, W, b, eps2d)
        z_next, gr = _tc_bn_pool(h, s1, s2, gamma, beta, onehot)
        return z_next, (z_next, gr)

    _, (zs, grs) = lax.scan(step, x_split, (Ws, bs, epss, gammas, betas))

    z_out = jnp.concatenate([zs[0, 0], zs[0, 1], zs[1, 0], zs[1, 1]], axis=1)
    g_out = jnp.concatenate([grs[0], grs[1]], axis=1)
    return (z_out, g_out)


# speedup vs baseline: 1.0380x; 1.0296x over previous
import functools

import jax
import jax.numpy as jnp
from jax import lax
from jax.experimental import pallas as pl
from jax.experimental.pallas import tpu as pltpu
from jax.experimental.pallas import tpu_sc as plsc

N = 10000
E = 320000
D = 128
DH = D // 2
G = 64
NL = 2
NC = 2
NS = 16
CHUNK = 128
CPW = 160
E_PAD = NS * CPW * CHUNK
AGG_ROWS = 10240
ZRPT = AGG_ROWS // NS
NBUF = 4


def _sc_segment_sum(z_cat, src_c, dst_c):
    mesh = plsc.VectorSubcoreMesh(core_axis_name="c", subcore_axis_name="s")

    @functools.partial(
        pl.kernel,
        out_type=jax.ShapeDtypeStruct((NC, AGG_ROWS, DH), jnp.float32),
        mesh=mesh,
        compiler_params=pltpu.CompilerParams(use_tc_tiling_on_sc=False),
        scratch_types=[
            pltpu.VMEM((CPW, CHUNK), jnp.int32),
            pltpu.VMEM((CPW, CHUNK), jnp.int32),
            pltpu.VMEM((NBUF, CHUNK, DH), jnp.float32),
            pltpu.VMEM_SHARED((AGG_ROWS, DH), jnp.float32),
            [pltpu.SemaphoreType.DMA] * NBUF,
            [pltpu.SemaphoreType.DMA] * NBUF,
        ],
    )
    def k(z_hbm, src_hbm, dst_hbm, out_hbm, src_v, dst_v, rows_v, agg_sh,
          gsems, ssems):
        c = lax.axis_index("c")
        s = lax.axis_index("s")

        pltpu.sync_copy(src_hbm.at[c, s], src_v)
        pltpu.sync_copy(dst_hbm.at[s], dst_v)

        zero = jnp.zeros((16,), jnp.float32)

        def zrow(i, carry):
            for j in range(DH // 16):
                rows_v[0, i, pl.ds(j * 16, 16)] = zero
            return carry

        lax.fori_loop(0, CHUNK, zrow, 0)
        for off in range(0, ZRPT, CHUNK):
            pltpu.sync_copy(rows_v.at[0],
                            agg_sh.at[pl.ds(s * ZRPT + off, CHUNK)])
        plsc.subcore_barrier()

        def gather(j, b):
            pltpu.async_copy(z_hbm.at[src_v.at[j]], rows_v.at[b], gsems[b])

        def gather_wait(j, b):
            pltpu.make_async_copy(
                z_hbm.at[src_v.at[j]], rows_v.at[b], gsems[b]).wait()

        def scatter(j, b):
            pltpu.async_copy(
                rows_v.at[b], agg_sh.at[dst_v.at[j]], ssems[b], add=True)

        def scatter_wait(j, b):
            pltpu.make_async_copy(
                rows_v.at[b], agg_sh.at[dst_v.at[j]], ssems[b]).wait()

        for b in range(NBUF):
            gather(b, b)
        n_it = CPW // NBUF

        def body(i, carry):
            for b in range(NBUF):
                j = i * NBUF + b
                gather_wait(j, b)
                scatter(j, b)
                scatter_wait(j, b)

                @pl.when(i < n_it - 1)
                def _():
                    gather(j + NBUF, b)
            return carry

        lax.fori_loop(0, n_it, body, 0)
        plsc.subcore_barrier()

        for off in range(0, ZRPT, CHUNK):
            base = s * ZRPT + off
            pltpu.sync_copy(agg_sh.at[pl.ds(base, CHUNK)], rows_v.at[0])
            pltpu.sync_copy(rows_v.at[0], out_hbm.at[c, pl.ds(base, CHUNK)])

    return k(z_cat, src_c, dst_c)


BLK = 2000


def _tc_linear_stats(z, p, W, b, eps2d):
    def body(z_ref, p_ref, w_ref, b_ref, e_ref, h_ref, s1_ref, s2_ref):
        i = pl.program_id(0)
        zb = jnp.concatenate([z_ref[0], z_ref[1]], axis=1)
        pb = jnp.concatenate([p_ref[0], p_ref[1]], axis=1)
        pre = (1.0 + e_ref[0, 0]) * zb + pb
        h = lax.dot_general(pre, w_ref[...], (((1,), (1,)), ((), ())),
                            precision=lax.Precision.HIGHEST,
                            preferred_element_type=jnp.float32)
        h = jnp.maximum(h + b_ref[...], 0.0)
        h_ref[...] = h

        @pl.when(i == 0)
        def _():
            s1_ref[...] = jnp.zeros_like(s1_ref)
            s2_ref[...] = jnp.zeros_like(s2_ref)

        s1_ref[...] += jnp.sum(h, axis=0)[None]
        s2_ref[...] += jnp.sum(h * h, axis=0)[None]

    return pl.pallas_call(
        body,
        grid=(N // BLK,),
        in_specs=[
            pl.BlockSpec((2, BLK, DH), lambda i: (0, i, 0)),
            pl.BlockSpec((2, BLK, DH), lambda i: (0, i, 0)),
            pl.BlockSpec((D, D), lambda i: (0, 0)),
            pl.BlockSpec((1, D), lambda i: (0, 0)),
            pl.BlockSpec((1, 1), lambda i: (0, 0)),
        ],
        out_specs=[
            pl.BlockSpec((BLK, D), lambda i: (i, 0)),
            pl.BlockSpec((1, D), lambda i: (0, 0)),
            pl.BlockSpec((1, D), lambda i: (0, 0)),
        ],
        out_shape=[
            jax.ShapeDtypeStruct((N, D), jnp.float32),
            jax.ShapeDtypeStruct((1, D), jnp.float32),
            jax.ShapeDtypeStruct((1, D), jnp.float32),
        ],
    )(z, p, W, b, eps2d)


def _tc_bn_pool(h, s1, s2, gamma, beta, onehot):
    def body(h_ref, s1_ref, s2_ref, g_ref, be_ref, oh_ref, z_ref, gout_ref):
        i = pl.program_id(0)
        mean = s1_ref[...] / N
        var = s2_ref[...] / N - mean * mean
        invstd = lax.rsqrt(var + 1e-5)
        z = (h_ref[...] - mean) * (invstd * g_ref[...]) + be_ref[...]
        z_ref[0] = z[:, :DH]
        z_ref[1] = z[:, DH:]

        @pl.when(i == 0)
        def _():
            gout_ref[...] = jnp.zeros_like(gout_ref)

        gout_ref[...] += lax.dot_general(
            oh_ref[...], z, (((0,), (0,)), ((), ())),
            precision=lax.Precision.HIGHEST,
            preferred_element_type=jnp.float32)

    return pl.pallas_call(
        body,
        grid=(N // BLK,),
        in_specs=[
            pl.BlockSpec((BLK, D), lambda i: (i, 0)),
            pl.BlockSpec((1, D), lambda i: (0, 0)),
            pl.BlockSpec((1, D), lambda i: (0, 0)),
            pl.BlockSpec((1, D), lambda i: (0, 0)),
            pl.BlockSpec((1, D), lambda i: (0, 0)),
            pl.BlockSpec((BLK, G), lambda i: (i, 0)),
        ],
        out_specs=[
            pl.BlockSpec((2, BLK, DH), lambda i: (0, i, 0)),
            pl.BlockSpec((G, D), lambda i: (0, 0)),
        ],
        out_shape=[
            jax.ShapeDtypeStruct((2, N, DH), jnp.float32),
            jax.ShapeDtypeStruct((G, D), jnp.float32),
        ],
    )(h, s1, s2, gamma, beta, onehot)


def kernel(x, edge_index, batch, W0, b0, eps0, g0, be0, W1, b1, eps1, g1, be1):
    npad = E_PAD - E
    pad_src = (jnp.arange(npad, dtype=jnp.int32) * 7) % N
    pad_dst = N + (jnp.arange(npad, dtype=jnp.int32) % (AGG_ROWS - N))
    src_p = jnp.concatenate([edge_index[0], pad_src])
    src_c = jnp.stack([src_p, src_p + N]).reshape(NC, NS, CPW, CHUNK)
    dst_c = jnp.concatenate([edge_index[1], pad_dst]).reshape(NS, CPW, CHUNK)

    onehot = (batch[:, None] == jnp.arange(G, dtype=batch.dtype)[None, :]
              ).astype(jnp.float32)

    x_split = jnp.stack([x[:, :DH], x[:, DH:]])

    Ws = jnp.stack([W0, W1])
    bs = jnp.stack([b0[None], b1[None]])
    epss = jnp.stack([jnp.reshape(eps0, (1, 1)), jnp.reshape(eps1, (1, 1))])
    gammas = jnp.stack([g0[None], g1[None]])
    betas = jnp.stack([be0[None], be1[None]])

    def step(z, params):
        W, b, eps2d, gamma, beta = params
        p = _sc_segment_sum(z.reshape(NC * N, DH), src_c, dst_c)
        h, s1, s2 = _tc_linear_stats(z, p, W, b, eps2d)
        z_next, gr = _tc_bn_pool(h, s1, s2, gamma, beta, onehot)
        return z_next, (z_next, gr)

    _, (zs, grs) = lax.scan(step, x_split, (Ws, bs, epss, gammas, betas))

    z_out = jnp.concatenate([zs[0, 0], zs[0, 1], zs[1, 0], zs[1, 1]], axis=1)
    g_out = jnp.concatenate([grs[0], grs[1]], axis=1)
    return (z_out, g_out)


# speedup vs baseline: 1.0401x; 1.0020x over previous
import functools

import jax
import jax.numpy as jnp
from jax import lax
from jax.experimental import pallas as pl
from jax.experimental.pallas import tpu as pltpu
from jax.experimental.pallas import tpu_sc as plsc

N = 10000
E = 320000
D = 128
DH = D // 2
G = 64
NL = 2
NC = 2
NS = 16
CHUNK = 128
CPW = 160
E_PAD = NS * CPW * CHUNK
AGG_ROWS = 10240
ZRPT = AGG_ROWS // NS
NBUF = 4


def _sc_segment_sum(z_cat, src_c, dst_c):
    mesh = plsc.VectorSubcoreMesh(core_axis_name="c", subcore_axis_name="s")

    @functools.partial(
        pl.kernel,
        out_type=jax.ShapeDtypeStruct((NC, AGG_ROWS, DH), jnp.float32),
        mesh=mesh,
        compiler_params=pltpu.CompilerParams(use_tc_tiling_on_sc=False),
        scratch_types=[
            pltpu.VMEM((CPW, CHUNK), jnp.int32),
            pltpu.VMEM((CPW, CHUNK), jnp.int32),
            pltpu.VMEM((NBUF, CHUNK, DH), jnp.float32),
            pltpu.VMEM_SHARED((AGG_ROWS, DH), jnp.float32),
            [pltpu.SemaphoreType.DMA] * NBUF,
            [pltpu.SemaphoreType.DMA] * NBUF,
        ],
    )
    def k(z_hbm, src_hbm, dst_hbm, out_hbm, src_v, dst_v, rows_v, agg_sh,
          gsems, ssems):
        c = lax.axis_index("c")
        s = lax.axis_index("s")

        pltpu.sync_copy(src_hbm.at[c, s], src_v)
        pltpu.sync_copy(dst_hbm.at[s], dst_v)

        zero = jnp.zeros((16,), jnp.float32)

        def zrow(i, carry):
            for j in range(DH // 16):
                rows_v[0, i, pl.ds(j * 16, 16)] = zero
            return carry

        lax.fori_loop(0, CHUNK, zrow, 0)
        for off in range(0, ZRPT, CHUNK):
            pltpu.sync_copy(rows_v.at[0],
                            agg_sh.at[pl.ds(s * ZRPT + off, CHUNK)])
        plsc.subcore_barrier()

        def gather(j, b):
            pltpu.async_copy(z_hbm.at[src_v.at[j]], rows_v.at[b], gsems[b])

        def gather_wait(j, b):
            pltpu.make_async_copy(
                z_hbm.at[src_v.at[j]], rows_v.at[b], gsems[b]).wait()

        def scatter(j, b):
            pltpu.async_copy(
                rows_v.at[b], agg_sh.at[dst_v.at[j]], ssems[b], add=True)

        def scatter_wait(j, b):
            pltpu.make_async_copy(
                rows_v.at[b], agg_sh.at[dst_v.at[j]], ssems[b]).wait()

        for b in range(NBUF):
            gather(b, b)
        n_it = CPW // NBUF

        def body(i, carry):
            for b in range(NBUF):
                j = i * NBUF + b
                gather_wait(j, b)
                scatter(j, b)
                scatter_wait(j, b)

                @pl.when(i < n_it - 1)
                def _():
                    gather(j + NBUF, b)
            return carry

        lax.fori_loop(0, n_it, body, 0)
        plsc.subcore_barrier()

        for off in range(0, ZRPT, CHUNK):
            base = s * ZRPT + off
            pltpu.sync_copy(agg_sh.at[pl.ds(base, CHUNK)], rows_v.at[0])
            pltpu.sync_copy(rows_v.at[0], out_hbm.at[c, pl.ds(base, CHUNK)])

    return k(z_cat, src_c, dst_c)


BLK = 5000


def _tc_linear_stats(z, p, W, b, eps2d):
    def body(z_ref, p_ref, w_ref, b_ref, e_ref, h_ref, s1_ref, s2_ref):
        i = pl.program_id(0)
        zb = jnp.concatenate([z_ref[0], z_ref[1]], axis=1)
        pb = jnp.concatenate([p_ref[0], p_ref[1]], axis=1)
        pre = (1.0 + e_ref[0, 0]) * zb + pb
        h = lax.dot_general(pre, w_ref[...], (((1,), (1,)), ((), ())),
                            precision=lax.Precision.HIGHEST,
                            preferred_element_type=jnp.float32)
        h = jnp.maximum(h + b_ref[...], 0.0)
        h_ref[...] = h

        @pl.when(i == 0)
        def _():
            s1_ref[...] = jnp.zeros_like(s1_ref)
            s2_ref[...] = jnp.zeros_like(s2_ref)

        s1_ref[...] += jnp.sum(h, axis=0)[None]
        s2_ref[...] += jnp.sum(h * h, axis=0)[None]

    return pl.pallas_call(
        body,
        grid=(N // BLK,),
        in_specs=[
            pl.BlockSpec((2, BLK, DH), lambda i: (0, i, 0)),
            pl.BlockSpec((2, BLK, DH), lambda i: (0, i, 0)),
            pl.BlockSpec((D, D), lambda i: (0, 0)),
            pl.BlockSpec((1, D), lambda i: (0, 0)),
            pl.BlockSpec((1, 1), lambda i: (0, 0)),
        ],
        out_specs=[
            pl.BlockSpec((BLK, D), lambda i: (i, 0)),
            pl.BlockSpec((1, D), lambda i: (0, 0)),
            pl.BlockSpec((1, D), lambda i: (0, 0)),
        ],
        out_shape=[
            jax.ShapeDtypeStruct((N, D), jnp.float32),
            jax.ShapeDtypeStruct((1, D), jnp.float32),
            jax.ShapeDtypeStruct((1, D), jnp.float32),
        ],
    )(z, p, W, b, eps2d)


def _tc_bn_pool(h, s1, s2, gamma, beta, onehot):
    def body(h_ref, s1_ref, s2_ref, g_ref, be_ref, oh_ref, z_ref, gout_ref):
        i = pl.program_id(0)
        mean = s1_ref[...] / N
        var = s2_ref[...] / N - mean * mean
        invstd = lax.rsqrt(var + 1e-5)
        z = (h_ref[...] - mean) * (invstd * g_ref[...]) + be_ref[...]
        z_ref[0] = z[:, :DH]
        z_ref[1] = z[:, DH:]

        @pl.when(i == 0)
        def _():
            gout_ref[...] = jnp.zeros_like(gout_ref)

        gout_ref[...] += lax.dot_general(
            oh_ref[...], z, (((0,), (0,)), ((), ())),
            precision=lax.Precision.HIGHEST,
            preferred_element_type=jnp.float32)

    return pl.pallas_call(
        body,
        grid=(N // BLK,),
        in_specs=[
            pl.BlockSpec((BLK, D), lambda i: (i, 0)),
            pl.BlockSpec((1, D), lambda i: (0, 0)),
            pl.BlockSpec((1, D), lambda i: (0, 0)),
            pl.BlockSpec((1, D), lambda i: (0, 0)),
            pl.BlockSpec((1, D), lambda i: (0, 0)),
            pl.BlockSpec((BLK, G), lambda i: (i, 0)),
        ],
        out_specs=[
            pl.BlockSpec((2, BLK, DH), lambda i: (0, i, 0)),
            pl.BlockSpec((G, D), lambda i: (0, 0)),
        ],
        out_shape=[
            jax.ShapeDtypeStruct((2, N, DH), jnp.float32),
            jax.ShapeDtypeStruct((G, D), jnp.float32),
        ],
    )(h, s1, s2, gamma, beta, onehot)


def kernel(x, edge_index, batch, W0, b0, eps0, g0, be0, W1, b1, eps1, g1, be1):
    npad = E_PAD - E
    pad_src = (jnp.arange(npad, dtype=jnp.int32) * 7) % N
    pad_dst = N + (jnp.arange(npad, dtype=jnp.int32) % (AGG_ROWS - N))
    src_p = jnp.concatenate([edge_index[0], pad_src])
    src_c = jnp.stack([src_p, src_p + N]).reshape(NC, NS, CPW, CHUNK)
    dst_c = jnp.concatenate([edge_index[1], pad_dst]).reshape(NS, CPW, CHUNK)

    onehot = (batch[:, None] == jnp.arange(G, dtype=batch.dtype)[None, :]
              ).astype(jnp.float32)

    x_split = jnp.stack([x[:, :DH], x[:, DH:]])

    Ws = jnp.stack([W0, W1])
    bs = jnp.stack([b0[None], b1[None]])
    epss = jnp.stack([jnp.reshape(eps0, (1, 1)), jnp.reshape(eps1, (1, 1))])
    gammas = jnp.stack([g0[None], g1[None]])
    betas = jnp.stack([be0[None], be1[None]])

    def step(z, params):
        W, b, eps2d, gamma, beta = params
        p = _sc_segment_sum(z.reshape(NC * N, DH), src_c, dst_c)
        h, s1, s2 = _tc_linear_stats(z, p, W, b, eps2d)
        z_next, gr = _tc_bn_pool(h, s1, s2, gamma, beta, onehot)
        return z_next, (z_next, gr)

    _, (zs, grs) = lax.scan(step, x_split, (Ws, bs, epss, gammas, betas))

    z_out = jnp.concatenate([zs[0, 0], zs[0, 1], zs[1, 0], zs[1, 1]], axis=1)
    g_out = jnp.concatenate([grs[0], grs[1]], axis=1)
    return (z_out, g_out)
